# baseline (speedup 1.0000x reference)
import functools

import jax
import jax.numpy as jnp
from jax import lax
from jax.experimental import pallas as pl
from jax.experimental.pallas import tpu as pltpu
from jax.experimental.pallas import tpu_sc as plsc

_N = 8192
_H = 32
_L = 3
_CUTOFF = 0.05
_BR = 256
_NW = 32
_RSC = 1024
_RPW = _RSC // _NW


_BIG = 1e19


def _colsum_body(cb_ref, d_ref, out_ref, q_ref):
    g = pl.program_id(0)

    @pl.when(g == 0)
    def _init():
        out_ref[...] = jnp.zeros_like(out_ref)

    d = d_ref[...]
    q_ref[...] = jnp.where(d < _CUTOFF, 1.0 / d, _BIG)
    q = q_ref[...]
    ones = jnp.ones((1, _BR), jnp.bfloat16)
    for l in range(_L):
        s = q - cb_ref[0, l]
        w = jax.lax.exp2(((s * s) * cb_ref[1, l]).astype(jnp.bfloat16))
        out_ref[l : l + 1, :] += jnp.dot(
            ones, w, preferred_element_type=jnp.float32
        )

    dd = d_ref[:, pl.ds(g * _BR, _BR)]
    rl = jax.lax.broadcasted_iota(jnp.int32, (_BR, _BR), 0)
    cl = jax.lax.broadcasted_iota(jnp.int32, (_BR, _BR), 1)
    qd = jnp.where((rl == cl) & (dd < _CUTOFF), 1.0 / dd, _BIG)
    for l in range(_L):
        s = qd - cb_ref[0, l]
        w = jax.lax.exp2((s * s) * cb_ref[1, l])
        out_ref[l : l + 1, pl.ds(g * _BR, _BR)] -= jnp.sum(
            w, axis=0, keepdims=True
        )


def _sc_colsum_body(d_hbm, cb_hbm, out_hbm, cb_v, row_v, acc_v):
    wid = lax.axis_index("s") * 2 + lax.axis_index("c")
    pltpu.sync_copy(cb_hbm, cb_v)

    zeros16 = jnp.zeros((16,), jnp.float32)

    def zero_body(j, carry):
        for l in range(_L):
            acc_v[l, pl.ds(j * 16, 16)] = zeros16
        return carry

    lax.fori_loop(0, _N // 16, zero_body, 0)

    cv = [cb_v[0, l, :] for l in range(_L)]
    bv = [cb_v[1, l, :] for l in range(_L)]
    row0 = _N - _RSC + wid * _RPW

    def row_body(r, carry):
        i = row0 + r
        pltpu.sync_copy(d_hbm.at[i], row_v)

        def vec_body(j, c2):
            sl = pl.ds(j * 16, 16)
            dv = row_v[sl]
            q = jnp.where(dv < _CUTOFF, 1.0 / dv, _BIG)
            for l in range(_L):
                t = q - cv[l]
                acc_v[l, sl] = acc_v[l, sl] + jnp.exp((t * t) * bv[l])
            return c2

        lax.fori_loop(0, _N // 16, vec_body, 0)

        vbase = (i // 16) * 16
        sl = pl.ds(vbase, 16)
        dvec = row_v[sl]
        lanes = lax.iota(jnp.int32, 16)
        m = (lanes == (i % 16)) & (dvec < _CUTOFF)
        qd = jnp.where(m, 1.0 / dvec, _BIG)
        for l in range(_L):
            t = qd - cv[l]
            acc_v[l, sl] = acc_v[l, sl] - jnp.exp((t * t) * bv[l])
        return carry

    lax.fori_loop(0, _RPW, row_body, 0)
    pltpu.sync_copy(acc_v, out_hbm.at[wid])


def _head_body(z_ref, won_ref, bon_ref, wint_ref, bint_ref, cs_ref, scp_ref,
               wa_ref, bn_ref, ppp_ref, out_ppp_ref, out_reg_ref):
    z = z_ref[...]
    cs_all = cs_ref[...] + jnp.sum(scp_ref[...], axis=0)
    zcs = jnp.zeros((1, _H), jnp.float32)
    for l in range(_L):
        x = jnp.dot(z, won_ref[l], preferred_element_type=jnp.float32)
        x = x + bon_ref[l : l + 1, :]
        zos = jnp.maximum(x, 0.0) + jnp.log1p(jnp.exp(-jnp.abs(x)))
        s_on = jnp.sum(zos, axis=0, keepdims=True)
        cs = cs_all[l : l + 1, :]
        v = jnp.dot(cs, zos, preferred_element_type=jnp.float32)
        inter = jnp.dot(v, wint_ref[l], preferred_element_type=jnp.float32)
        inter = inter + jnp.sum(cs) * bint_ref[l : l + 1, :]
        zcs = zcs + wa_ref[l : l + 1, :] * (s_on + inter) + bn_ref[0, l]
    out_ppp_ref[...] = ppp_ref[...] + 0.01 * zcs
    out_reg_ref[...] = 0.01 * jnp.sqrt(jnp.sum(zcs * zcs)).reshape(1, 1)


def kernel(geom_array, dist_matrix, ppp_params, W_on, B_on, W_int, B_int,
           mu, sigma, W_a, B_n):
    n = dist_matrix.shape[0]
    log2e = 1.4426950408889634
    cb = jnp.stack(
        [1.0 / mu, -log2e / (2.0 * sigma * sigma)]
    ).astype(jnp.float32)

    colsums = pl.pallas_call(
        _colsum_body,
        grid=((n - _RSC) // _BR,),
        in_specs=[
            pl.BlockSpec(memory_space=pltpu.SMEM),
            pl.BlockSpec((_BR, n), lambda g: (g, 0)),
        ],
        out_specs=pl.BlockSpec((_L, n), lambda g: (0, 0)),
        out_shape=jax.ShapeDtypeStruct((_L, n), jnp.float32),
        scratch_shapes=[pltpu.VMEM((_BR, n), jnp.float32)],
    )(cb, dist_matrix)

    cb_nat = jnp.stack([1.0 / mu, -1.0 / (2.0 * sigma * sigma)])
    cb_sc = jnp.broadcast_to(
        cb_nat.astype(jnp.float32)[:, :, None], (2, _L, 16)
    )
    mesh = plsc.VectorSubcoreMesh(core_axis_name="c", subcore_axis_name="s")
    sc_parts = pl.kernel(
        _sc_colsum_body,
        out_type=jax.ShapeDtypeStruct((_NW, _L, n), jnp.float32),
        mesh=mesh,
        scratch_types=[
            pltpu.VMEM((2, _L, 16), jnp.float32),
            pltpu.VMEM((n,), jnp.float32),
            pltpu.VMEM((_L, n), jnp.float32),
        ],
    )(dist_matrix, cb_sc)

    out_ppp, out_reg = pl.pallas_call(
        _head_body,
        in_specs=[
            pl.BlockSpec(memory_space=pltpu.VMEM),
            pl.BlockSpec(memory_space=pltpu.VMEM),
            pl.BlockSpec(memory_space=pltpu.VMEM),
            pl.BlockSpec(memory_space=pltpu.VMEM),
            pl.BlockSpec(memory_space=pltpu.VMEM),
            pl.BlockSpec(memory_space=pltpu.VMEM),
            pl.BlockSpec(memory_space=pltpu.VMEM),
            pl.BlockSpec(memory_space=pltpu.VMEM),
            pl.BlockSpec(memory_space=pltpu.SMEM),
            pl.BlockSpec(memory_space=pltpu.VMEM),
        ],
        out_specs=[
            pl.BlockSpec(memory_space=pltpu.VMEM),
            pl.BlockSpec(memory_space=pltpu.VMEM),
        ],
        out_shape=[
            jax.ShapeDtypeStruct((1, _H), jnp.float32),
            jax.ShapeDtypeStruct((1, 1), jnp.float32),
        ],
    )(geom_array, W_on, B_on, W_int, B_int, colsums, sc_parts, W_a,
      B_n.reshape(1, _L), ppp_params.reshape(1, _H))

    return out_ppp.reshape(_H), out_reg.reshape(())


# speedup vs baseline: 1.0383x; 1.0383x over previous
import functools

import jax
import jax.numpy as jnp
from jax import lax
from jax.experimental import pallas as pl
from jax.experimental.pallas import tpu as pltpu
from jax.experimental.pallas import tpu_sc as plsc

_N = 8192
_H = 32
_L = 3
_CUTOFF = 0.05
_BR = 256
_NW = 32
_RSC = 1024
_RPW = _RSC // _NW


_BIG = 1e19


def _colsum_body(cb_ref, d_ref, out_ref, q_ref):
    g = pl.program_id(0)

    @pl.when(g == 0)
    def _init():
        out_ref[...] = jnp.zeros_like(out_ref)

    d = d_ref[...]
    q_ref[...] = jnp.where(d < _CUTOFF, 1.0 / d, _BIG)
    q = q_ref[...]
    ones = jnp.ones((1, _BR), jnp.bfloat16)
    for l in range(_L):
        s = q - cb_ref[0, l]
        w = jax.lax.exp2(((s * s) * cb_ref[1, l]).astype(jnp.bfloat16))
        out_ref[l : l + 1, :] += jnp.dot(
            ones, w, preferred_element_type=jnp.float32
        )

    dd = d_ref[:, pl.ds(g * _BR, _BR)]
    rl = jax.lax.broadcasted_iota(jnp.int32, (_BR, _BR), 0)
    cl = jax.lax.broadcasted_iota(jnp.int32, (_BR, _BR), 1)
    qd = jnp.where((rl == cl) & (dd < _CUTOFF), 1.0 / dd, _BIG)
    for l in range(_L):
        s = qd - cb_ref[0, l]
        w = jax.lax.exp2((s * s) * cb_ref[1, l])
        out_ref[l : l + 1, pl.ds(g * _BR, _BR)] -= jnp.sum(
            w, axis=0, keepdims=True
        )


def _sc_colsum_body(d_hbm, cb_hbm, out_hbm, cb_v, row_v, acc_v):
    wid = lax.axis_index("s") * 2 + lax.axis_index("c")
    pltpu.sync_copy(cb_hbm, cb_v)

    zeros16 = jnp.zeros((16,), jnp.float32)

    def zero_body(j, carry):
        for l in range(_L):
            acc_v[l, pl.ds(j * 16, 16)] = zeros16
        return carry

    lax.fori_loop(0, _N // 16, zero_body, 0)

    cv = [cb_v[0, l, :] for l in range(_L)]
    bv = [cb_v[1, l, :] for l in range(_L)]
    row0 = _N - _RSC + wid * _RPW

    def row_body(r, carry):
        i = row0 + r
        pltpu.sync_copy(d_hbm.at[i], row_v)

        def vec_body(j, c2):
            for k in range(4):
                sl = pl.ds(j * 64 + k * 16, 16)
                dv = row_v[sl]
                q = jnp.where(dv < _CUTOFF, 1.0 / dv, _BIG)
                for l in range(_L):
                    t = q - cv[l]
                    plsc.addupdate(acc_v.at[l, sl], jnp.exp((t * t) * bv[l]))
            return c2

        lax.fori_loop(0, _N // 64, vec_body, 0)

        vbase = (i // 16) * 16
        sl = pl.ds(vbase, 16)
        dvec = row_v[sl]
        lanes = lax.iota(jnp.int32, 16)
        m = (lanes == (i % 16)) & (dvec < _CUTOFF)
        qd = jnp.where(m, 1.0 / dvec, _BIG)
        for l in range(_L):
            t = qd - cv[l]
            plsc.addupdate(acc_v.at[l, sl], -jnp.exp((t * t) * bv[l]))
        return carry

    lax.fori_loop(0, _RPW, row_body, 0)
    pltpu.sync_copy(acc_v, out_hbm.at[wid])


def _head_body(z_ref, won_ref, bon_ref, wint_ref, bint_ref, cs_ref, scp_ref,
               wa_ref, bn_ref, ppp_ref, out_ppp_ref, out_reg_ref):
    z = z_ref[...]
    cs_all = cs_ref[...] + jnp.sum(scp_ref[...], axis=0)
    zcs = jnp.zeros((1, _H), jnp.float32)
    for l in range(_L):
        x = jnp.dot(z, won_ref[l], preferred_element_type=jnp.float32)
        x = x + bon_ref[l : l + 1, :]
        zos = jnp.maximum(x, 0.0) + jnp.log1p(jnp.exp(-jnp.abs(x)))
        s_on = jnp.sum(zos, axis=0, keepdims=True)
        cs = cs_all[l : l + 1, :]
        v = jnp.dot(cs, zos, preferred_element_type=jnp.float32)
        inter = jnp.dot(v, wint_ref[l], preferred_element_type=jnp.float32)
        inter = inter + jnp.sum(cs) * bint_ref[l : l + 1, :]
        zcs = zcs + wa_ref[l : l + 1, :] * (s_on + inter) + bn_ref[0, l]
    out_ppp_ref[...] = ppp_ref[...] + 0.01 * zcs
    out_reg_ref[...] = 0.01 * jnp.sqrt(jnp.sum(zcs * zcs)).reshape(1, 1)


def kernel(geom_array, dist_matrix, ppp_params, W_on, B_on, W_int, B_int,
           mu, sigma, W_a, B_n):
    n = dist_matrix.shape[0]
    log2e = 1.4426950408889634
    cb = jnp.stack(
        [1.0 / mu, -log2e / (2.0 * sigma * sigma)]
    ).astype(jnp.float32)

    colsums = pl.pallas_call(
        _colsum_body,
        grid=((n - _RSC) // _BR,),
        in_specs=[
            pl.BlockSpec(memory_space=pltpu.SMEM),
            pl.BlockSpec((_BR, n), lambda g: (g, 0)),
        ],
        out_specs=pl.BlockSpec((_L, n), lambda g: (0, 0)),
        out_shape=jax.ShapeDtypeStruct((_L, n), jnp.float32),
        scratch_shapes=[pltpu.VMEM((_BR, n), jnp.float32)],
    )(cb, dist_matrix)

    cb_nat = jnp.stack([1.0 / mu, -1.0 / (2.0 * sigma * sigma)])
    cb_sc = jnp.broadcast_to(
        cb_nat.astype(jnp.float32)[:, :, None], (2, _L, 16)
    )
    mesh = plsc.VectorSubcoreMesh(core_axis_name="c", subcore_axis_name="s")
    sc_parts = pl.kernel(
        _sc_colsum_body,
        out_type=jax.ShapeDtypeStruct((_NW, _L, n), jnp.float32),
        mesh=mesh,
        scratch_types=[
            pltpu.VMEM((2, _L, 16), jnp.float32),
            pltpu.VMEM((n,), jnp.float32),
            pltpu.VMEM((_L, n), jnp.float32),
        ],
    )(dist_matrix, cb_sc)

    out_ppp, out_reg = pl.pallas_call(
        _head_body,
        in_specs=[
            pl.BlockSpec(memory_space=pltpu.VMEM),
            pl.BlockSpec(memory_space=pltpu.VMEM),
            pl.BlockSpec(memory_space=pltpu.VMEM),
            pl.BlockSpec(memory_space=pltpu.VMEM),
            pl.BlockSpec(memory_space=pltpu.VMEM),
            pl.BlockSpec(memory_space=pltpu.VMEM),
            pl.BlockSpec(memory_space=pltpu.VMEM),
            pl.BlockSpec(memory_space=pltpu.VMEM),
            pl.BlockSpec(memory_space=pltpu.SMEM),
            pl.BlockSpec(memory_space=pltpu.VMEM),
        ],
        out_specs=[
            pl.BlockSpec(memory_space=pltpu.VMEM),
            pl.BlockSpec(memory_space=pltpu.VMEM),
        ],
        out_shape=[
            jax.ShapeDtypeStruct((1, _H), jnp.float32),
            jax.ShapeDtypeStruct((1, 1), jnp.float32),
        ],
    )(geom_array, W_on, B_on, W_int, B_int, colsums, sc_parts, W_a,
      B_n.reshape(1, _L), ppp_params.reshape(1, _H))

    return out_ppp.reshape(_H), out_reg.reshape(())


# speedup vs baseline: 2.4825x; 2.3910x over previous
import functools

import jax
import jax.numpy as jnp
from jax import lax
from jax.experimental import pallas as pl
from jax.experimental.pallas import tpu as pltpu
from jax.experimental.pallas import tpu_sc as plsc

_N = 8192
_H = 32
_L = 3
_CUTOFF = 0.05
_BR = 256
_NW = 32
_RSC = 1024
_RPW = _RSC // _NW


_BIG = 1e19


def _colsum_body(cb_ref, d_ref, out_ref, q_ref):
    g = pl.program_id(0)

    @pl.when(g == 0)
    def _init():
        out_ref[...] = jnp.zeros_like(out_ref)

    d = d_ref[...]
    q_ref[...] = jnp.where(d < _CUTOFF, 1.0 / d, _BIG)
    q = q_ref[...]
    ones = jnp.ones((1, _BR), jnp.bfloat16)
    for l in range(_L):
        s = q - cb_ref[0, l]
        w = jax.lax.exp2(((s * s) * cb_ref[1, l]).astype(jnp.bfloat16))
        out_ref[l : l + 1, :] += jnp.dot(
            ones, w, preferred_element_type=jnp.float32
        )

    dd = d_ref[:, pl.ds(g * _BR, _BR)]
    rl = jax.lax.broadcasted_iota(jnp.int32, (_BR, _BR), 0)
    cl = jax.lax.broadcasted_iota(jnp.int32, (_BR, _BR), 1)
    qd = jnp.where((rl == cl) & (dd < _CUTOFF), 1.0 / dd, _BIG)
    for l in range(_L):
        s = qd - cb_ref[0, l]
        w = jax.lax.exp2((s * s) * cb_ref[1, l])
        out_ref[l : l + 1, pl.ds(g * _BR, _BR)] -= jnp.sum(
            w, axis=0, keepdims=True
        )


def _sc_colsum_body(d_hbm, cb_hbm, out_hbm, cb_v, row_v, acc_v):
    wid = lax.axis_index("s") * 2 + lax.axis_index("c")
    pltpu.sync_copy(cb_hbm, cb_v)

    zeros16 = jnp.zeros((16,), jnp.float32)

    def zero_body(j, carry):
        for l in range(_L):
            acc_v[l, pl.ds(j * 16, 16)] = zeros16
        return carry

    lax.fori_loop(0, _N // 16, zero_body, 0)

    cv = [cb_v[0, l, :] for l in range(_L)]
    bv = [cb_v[1, l, :] for l in range(_L)]
    row0 = _N - _RSC + wid * _RPW

    def row_body(r, carry):
        i = row0 + r
        pltpu.sync_copy(d_hbm.at[i], row_v)

        @plsc.parallel_loop(0, _N // 16, step=1, unroll=8)
        def vec_body(j):
            sl = pl.ds(j * 16, 16)
            dv = row_v[sl]
            q = jnp.where(dv < _CUTOFF, 1.0 / dv, _BIG)
            for l in range(_L):
                t = q - cv[l]
                plsc.addupdate(acc_v.at[l, sl], jnp.exp((t * t) * bv[l]))

        vbase = (i // 16) * 16
        sl = pl.ds(vbase, 16)
        dvec = row_v[sl]
        lanes = lax.iota(jnp.int32, 16)
        m = (lanes == (i % 16)) & (dvec < _CUTOFF)
        qd = jnp.where(m, 1.0 / dvec, _BIG)
        for l in range(_L):
            t = qd - cv[l]
            plsc.addupdate(acc_v.at[l, sl], -jnp.exp((t * t) * bv[l]))
        return carry

    lax.fori_loop(0, _RPW, row_body, 0)
    pltpu.sync_copy(acc_v, out_hbm.at[wid])


def _head_body(z_ref, won_ref, bon_ref, wint_ref, bint_ref, cs_ref, scp_ref,
               wa_ref, bn_ref, ppp_ref, out_ppp_ref, out_reg_ref):
    z = z_ref[...]
    cs_all = cs_ref[...] + jnp.sum(scp_ref[...], axis=0)
    zcs = jnp.zeros((1, _H), jnp.float32)
    for l in range(_L):
        x = jnp.dot(z, won_ref[l], preferred_element_type=jnp.float32)
        x = x + bon_ref[l : l + 1, :]
        zos = jnp.maximum(x, 0.0) + jnp.log1p(jnp.exp(-jnp.abs(x)))
        s_on = jnp.sum(zos, axis=0, keepdims=True)
        cs = cs_all[l : l + 1, :]
        v = jnp.dot(cs, zos, preferred_element_type=jnp.float32)
        inter = jnp.dot(v, wint_ref[l], preferred_element_type=jnp.float32)
        inter = inter + jnp.sum(cs) * bint_ref[l : l + 1, :]
        zcs = zcs + wa_ref[l : l + 1, :] * (s_on + inter) + bn_ref[0, l]
    out_ppp_ref[...] = ppp_ref[...] + 0.01 * zcs
    out_reg_ref[...] = 0.01 * jnp.sqrt(jnp.sum(zcs * zcs)).reshape(1, 1)


def kernel(geom_array, dist_matrix, ppp_params, W_on, B_on, W_int, B_int,
           mu, sigma, W_a, B_n):
    n = dist_matrix.shape[0]
    log2e = 1.4426950408889634
    cb = jnp.stack(
        [1.0 / mu, -log2e / (2.0 * sigma * sigma)]
    ).astype(jnp.float32)

    colsums = pl.pallas_call(
        _colsum_body,
        grid=((n - _RSC) // _BR,),
        in_specs=[
            pl.BlockSpec(memory_space=pltpu.SMEM),
            pl.BlockSpec((_BR, n), lambda g: (g, 0)),
        ],
        out_specs=pl.BlockSpec((_L, n), lambda g: (0, 0)),
        out_shape=jax.ShapeDtypeStruct((_L, n), jnp.float32),
        scratch_shapes=[pltpu.VMEM((_BR, n), jnp.float32)],
    )(cb, dist_matrix)

    cb_nat = jnp.stack([1.0 / mu, -1.0 / (2.0 * sigma * sigma)])
    cb_sc = jnp.broadcast_to(
        cb_nat.astype(jnp.float32)[:, :, None], (2, _L, 16)
    )
    mesh = plsc.VectorSubcoreMesh(core_axis_name="c", subcore_axis_name="s")
    sc_parts = pl.kernel(
        _sc_colsum_body,
        out_type=jax.ShapeDtypeStruct((_NW, _L, n), jnp.float32),
        mesh=mesh,
        scratch_types=[
            pltpu.VMEM((2, _L, 16), jnp.float32),
            pltpu.VMEM((n,), jnp.float32),
            pltpu.VMEM((_L, n), jnp.float32),
        ],
    )(dist_matrix, cb_sc)

    out_ppp, out_reg = pl.pallas_call(
        _head_body,
        in_specs=[
            pl.BlockSpec(memory_space=pltpu.VMEM),
            pl.BlockSpec(memory_space=pltpu.VMEM),
            pl.BlockSpec(memory_space=pltpu.VMEM),
            pl.BlockSpec(memory_space=pltpu.VMEM),
            pl.BlockSpec(memory_space=pltpu.VMEM),
            pl.BlockSpec(memory_space=pltpu.VMEM),
            pl.BlockSpec(memory_space=pltpu.VMEM),
            pl.BlockSpec(memory_space=pltpu.VMEM),
            pl.BlockSpec(memory_space=pltpu.SMEM),
            pl.BlockSpec(memory_space=pltpu.VMEM),
        ],
        out_specs=[
            pl.BlockSpec(memory_space=pltpu.VMEM),
            pl.BlockSpec(memory_space=pltpu.VMEM),
        ],
        out_shape=[
            jax.ShapeDtypeStruct((1, _H), jnp.float32),
            jax.ShapeDtypeStruct((1, 1), jnp.float32),
        ],
    )(geom_array, W_on, B_on, W_int, B_int, colsums, sc_parts, W_a,
      B_n.reshape(1, _L), ppp_params.reshape(1, _H))

    return out_ppp.reshape(_H), out_reg.reshape(())
